# baseline (speedup 1.0000x reference)
import functools

import jax
import jax.numpy as jnp
from jax import lax
from jax.experimental import pallas as pl
from jax.experimental.pallas import tpu as pltpu
from jax.experimental.pallas import tpu_sc as plsc

D_MODEL = 128
M_VOCAB = 1000
SEQ_LEN = 4096
BETA = 0.5

_NC, _NS, _L = 2, 16, 16
_NW = _NC * _NS
_BPW = SEQ_LEN // _NW
_DCH = D_MODEL // _L
_NH = 8
_HPOS = _BPW // _NH
_M_PAD = 1024
_ROWS_PER_TILE = _M_PAD // _NS


def _sc_body(t_hbm, idx_hbm, table_hbm, wt_hbm, bt_hbm, out_hbm,
             idx_v, t_v, rows_v, wt_v, bt_v, table_sp, gsem, wsem, ssem):
    sid = lax.axis_index("s")
    wid = sid * _NC + lax.axis_index("c")
    base = wid * _BPW

    stage = pltpu.async_copy(
        table_hbm.at[pl.ds(sid * _ROWS_PER_TILE, _ROWS_PER_TILE)],
        table_sp.at[pl.ds(sid * _ROWS_PER_TILE, _ROWS_PER_TILE)],
        ssem.at[3],
    )
    pltpu.sync_copy(idx_hbm.at[pl.ds(base, _BPW)], idx_v)
    small = [
        pltpu.async_copy(t_hbm.at[pl.ds(base, _BPW)], t_v, ssem.at[0]),
        pltpu.async_copy(wt_hbm, wt_v, ssem.at[1]),
        pltpu.async_copy(bt_hbm, bt_v, ssem.at[2]),
    ]
    stage.wait()
    plsc.subcore_barrier()
    gathers = [
        pltpu.async_copy(
            table_sp.at[idx_v.at[pl.ds(h * _HPOS, _HPOS)]],
            rows_v.at[pl.ds(h * _HPOS, _HPOS)],
            gsem.at[h],
        )
        for h in range(_NH)
    ]
    for s in small:
        s.wait()
    wt = [wt_v[pl.ds(dc * _L, _L)] for dc in range(_DCH)]
    bt = [bt_v[pl.ds(dc * _L, _L)] for dc in range(_DCH)]

    writes = []
    for h in range(_NH):
        gathers[h].wait()

        def g_step(g, _, h=h):
            p0 = h * _HPOS + g * _L
            t16 = t_v[pl.ds(p0, _L)]
            fac16 = jnp.where(t16 < 0.0, 0.0, BETA)
            for j in range(_L):
                s = p0 + j
                ts = jnp.full((_L,), t16[j])
                fac = jnp.full((_L,), fac16[j])
                for dc in range(_DCH):
                    sl = pl.ds(dc * _L, _L)
                    te = ts * wt[dc] + bt[dc]
                    rows_v[s, sl] = fac * (rows_v[s, sl] + te)
            return 0

        lax.fori_loop(0, _HPOS // _L, g_step, 0)
        writes.append(pltpu.async_copy(
            rows_v.at[pl.ds(h * _HPOS, _HPOS)],
            out_hbm.at[pl.ds(base + h * _HPOS, _HPOS)],
            wsem.at[h],
        ))
    for w in writes:
        w.wait()
    plsc.subcore_barrier()


@functools.partial(
    pl.kernel,
    mesh=plsc.VectorSubcoreMesh(core_axis_name="c", subcore_axis_name="s"),
    out_type=jax.ShapeDtypeStruct((SEQ_LEN, D_MODEL), jnp.float32),
    scratch_types=[
        pltpu.VMEM((_BPW,), jnp.int32),
        pltpu.VMEM((_BPW,), jnp.float32),
        pltpu.VMEM((_BPW, D_MODEL), jnp.float32),
        pltpu.VMEM((D_MODEL,), jnp.float32),
        pltpu.VMEM((D_MODEL,), jnp.float32),
        pltpu.MemorySpace.VMEM_SHARED((_M_PAD, D_MODEL), jnp.float32),
        pltpu.SemaphoreType.DMA((_NH,)),
        pltpu.SemaphoreType.DMA((_NH,)),
        pltpu.SemaphoreType.DMA((4,)),
    ],
)
def _sc_embed(t_hbm, idx_hbm, table_hbm, wt_hbm, bt_hbm, out_hbm,
              idx_v, t_v, rows_v, wt_v, bt_v, table_sp, gsem, wsem, ssem):
    _sc_body(t_hbm, idx_hbm, table_hbm, wt_hbm, bt_hbm, out_hbm,
             idx_v, t_v, rows_v, wt_v, bt_v, table_sp, gsem, wsem, ssem)


def kernel(x, W_m, W_t, b_t):
    t = x[:, 0]
    idx = x[:, 1].astype(jnp.int32)
    table = jnp.pad(W_m.T, ((0, _M_PAD - M_VOCAB), (0, 0)))
    return _sc_embed(t, idx, table, W_t, b_t)


# speedup vs baseline: 1.0821x; 1.0821x over previous
import functools

import jax
import jax.numpy as jnp
from jax import lax
from jax.experimental import pallas as pl
from jax.experimental.pallas import tpu as pltpu
from jax.experimental.pallas import tpu_sc as plsc

D_MODEL = 128
M_VOCAB = 1000
SEQ_LEN = 4096
BETA = 0.5

_NC, _NS, _L = 2, 16, 16
_NW = _NC * _NS
_BPW = SEQ_LEN // _NW
_DCH = D_MODEL // _L
_NH = 2
_HPOS = _BPW // _NH
_M_PAD = 1024
_ROWS_PER_TILE = _M_PAD // _NS


def _sc_body(t_hbm, idx_hbm, table_hbm, wt_hbm, bt_hbm, out_hbm,
             idx_v, t_v, rows_v, wt_v, bt_v, table_sp, gsem, wsem, ssem):
    sid = lax.axis_index("s")
    wid = sid * _NC + lax.axis_index("c")
    base = wid * _BPW

    stage = pltpu.async_copy(
        table_hbm.at[pl.ds(sid * _ROWS_PER_TILE, _ROWS_PER_TILE)],
        table_sp.at[pl.ds(sid * _ROWS_PER_TILE, _ROWS_PER_TILE)],
        ssem.at[3],
    )
    pltpu.sync_copy(idx_hbm.at[pl.ds(base, _BPW)], idx_v)
    small = [
        pltpu.async_copy(t_hbm.at[pl.ds(base, _BPW)], t_v, ssem.at[0]),
        pltpu.async_copy(wt_hbm, wt_v, ssem.at[1]),
        pltpu.async_copy(bt_hbm, bt_v, ssem.at[2]),
    ]
    stage.wait()
    plsc.subcore_barrier()
    gathers = [
        pltpu.async_copy(
            table_sp.at[idx_v.at[pl.ds(h * _HPOS, _HPOS)]],
            rows_v.at[pl.ds(h * _HPOS, _HPOS)],
            gsem.at[h],
        )
        for h in range(_NH)
    ]
    for s in small:
        s.wait()
    wt = [wt_v[pl.ds(dc * _L, _L)] for dc in range(_DCH)]
    bt = [bt_v[pl.ds(dc * _L, _L)] for dc in range(_DCH)]

    writes = []
    for h in range(_NH):
        gathers[h].wait()

        def g_step(g, _, h=h):
            p0 = h * _HPOS + g * _L
            t16 = t_v[pl.ds(p0, _L)]
            fac16 = jnp.where(t16 < 0.0, 0.0, BETA)
            for j in range(_L):
                s = p0 + j
                ts = jnp.full((_L,), t16[j])
                fac = jnp.full((_L,), fac16[j])
                for dc in range(_DCH):
                    sl = pl.ds(dc * _L, _L)
                    te = ts * wt[dc] + bt[dc]
                    rows_v[s, sl] = fac * (rows_v[s, sl] + te)
            return 0

        lax.fori_loop(0, _HPOS // _L, g_step, 0)
        writes.append(pltpu.async_copy(
            rows_v.at[pl.ds(h * _HPOS, _HPOS)],
            out_hbm.at[pl.ds(base + h * _HPOS, _HPOS)],
            wsem.at[h],
        ))
    for w in writes:
        w.wait()
    plsc.subcore_barrier()


@functools.partial(
    pl.kernel,
    mesh=plsc.VectorSubcoreMesh(core_axis_name="c", subcore_axis_name="s"),
    out_type=jax.ShapeDtypeStruct((SEQ_LEN, D_MODEL), jnp.float32),
    scratch_types=[
        pltpu.VMEM((_BPW,), jnp.int32),
        pltpu.VMEM((_BPW,), jnp.float32),
        pltpu.VMEM((_BPW, D_MODEL), jnp.float32),
        pltpu.VMEM((D_MODEL,), jnp.float32),
        pltpu.VMEM((D_MODEL,), jnp.float32),
        pltpu.MemorySpace.VMEM_SHARED((_M_PAD, D_MODEL), jnp.float32),
        pltpu.SemaphoreType.DMA((_NH,)),
        pltpu.SemaphoreType.DMA((_NH,)),
        pltpu.SemaphoreType.DMA((4,)),
    ],
)
def _sc_embed(t_hbm, idx_hbm, table_hbm, wt_hbm, bt_hbm, out_hbm,
              idx_v, t_v, rows_v, wt_v, bt_v, table_sp, gsem, wsem, ssem):
    _sc_body(t_hbm, idx_hbm, table_hbm, wt_hbm, bt_hbm, out_hbm,
             idx_v, t_v, rows_v, wt_v, bt_v, table_sp, gsem, wsem, ssem)


def kernel(x, W_m, W_t, b_t):
    t = x[:, 0]
    idx = x[:, 1].astype(jnp.int32)
    table = jnp.pad(W_m.T, ((0, _M_PAD - M_VOCAB), (0, 0)))
    return _sc_embed(t, idx, table, W_t, b_t)


# speedup vs baseline: 1.0831x; 1.0010x over previous
import functools

import jax
import jax.numpy as jnp
from jax import lax
from jax.experimental import pallas as pl
from jax.experimental.pallas import tpu as pltpu
from jax.experimental.pallas import tpu_sc as plsc

D_MODEL = 128
M_VOCAB = 1000
SEQ_LEN = 4096
BETA = 0.5

_NC, _NS, _L = 2, 16, 16
_NW = _NC * _NS
_BPW = SEQ_LEN // _NW
_DCH = D_MODEL // _L
_NH = 4
_HPOS = _BPW // _NH
_M_PAD = 1024
_ROWS_PER_TILE = _M_PAD // _NS


def _sc_body(t_hbm, idx_hbm, table_hbm, wt_hbm, bt_hbm, out_hbm,
             idx_v, t_v, rows_v, wt_v, bt_v, table_sp, gsem, wsem, ssem):
    sid = lax.axis_index("s")
    wid = sid * _NC + lax.axis_index("c")
    base = wid * _BPW

    stage = pltpu.async_copy(
        table_hbm.at[pl.ds(sid * _ROWS_PER_TILE, _ROWS_PER_TILE)],
        table_sp.at[pl.ds(sid * _ROWS_PER_TILE, _ROWS_PER_TILE)],
        ssem.at[3],
    )
    pltpu.sync_copy(idx_hbm.at[pl.ds(base, _BPW)], idx_v)
    small = [
        pltpu.async_copy(t_hbm.at[pl.ds(base, _BPW)], t_v, ssem.at[0]),
        pltpu.async_copy(wt_hbm, wt_v, ssem.at[1]),
        pltpu.async_copy(bt_hbm, bt_v, ssem.at[2]),
    ]
    stage.wait()
    plsc.subcore_barrier()
    gathers = [
        pltpu.async_copy(
            table_sp.at[idx_v.at[pl.ds(h * _HPOS, _HPOS)]],
            rows_v.at[pl.ds(h * _HPOS, _HPOS)],
            gsem.at[h],
        )
        for h in range(_NH)
    ]
    for s in small:
        s.wait()
    wt = [wt_v[pl.ds(dc * _L, _L)] for dc in range(_DCH)]
    bt = [bt_v[pl.ds(dc * _L, _L)] for dc in range(_DCH)]

    writes = []
    for h in range(_NH):
        gathers[h].wait()

        def g_step(g, _, h=h):
            p0 = h * _HPOS + g * _L
            t16 = t_v[pl.ds(p0, _L)]
            fac16 = jnp.where(t16 < 0.0, 0.0, BETA)
            for j in range(_L):
                s = p0 + j
                ts = jnp.full((_L,), t16[j])
                fac = jnp.full((_L,), fac16[j])
                for dc in range(_DCH):
                    sl = pl.ds(dc * _L, _L)
                    te = ts * wt[dc] + bt[dc]
                    rows_v[s, sl] = fac * (rows_v[s, sl] + te)
            return 0

        lax.fori_loop(0, _HPOS // _L, g_step, 0)
        writes.append(pltpu.async_copy(
            rows_v.at[pl.ds(h * _HPOS, _HPOS)],
            out_hbm.at[pl.ds(base + h * _HPOS, _HPOS)],
            wsem.at[h],
        ))
    for w in writes:
        w.wait()
    plsc.subcore_barrier()


@functools.partial(
    pl.kernel,
    mesh=plsc.VectorSubcoreMesh(core_axis_name="c", subcore_axis_name="s"),
    out_type=jax.ShapeDtypeStruct((SEQ_LEN, D_MODEL), jnp.float32),
    scratch_types=[
        pltpu.VMEM((_BPW,), jnp.int32),
        pltpu.VMEM((_BPW,), jnp.float32),
        pltpu.VMEM((_BPW, D_MODEL), jnp.float32),
        pltpu.VMEM((D_MODEL,), jnp.float32),
        pltpu.VMEM((D_MODEL,), jnp.float32),
        pltpu.MemorySpace.VMEM_SHARED((_M_PAD, D_MODEL), jnp.float32),
        pltpu.SemaphoreType.DMA((_NH,)),
        pltpu.SemaphoreType.DMA((_NH,)),
        pltpu.SemaphoreType.DMA((4,)),
    ],
)
def _sc_embed(t_hbm, idx_hbm, table_hbm, wt_hbm, bt_hbm, out_hbm,
              idx_v, t_v, rows_v, wt_v, bt_v, table_sp, gsem, wsem, ssem):
    _sc_body(t_hbm, idx_hbm, table_hbm, wt_hbm, bt_hbm, out_hbm,
             idx_v, t_v, rows_v, wt_v, bt_v, table_sp, gsem, wsem, ssem)


def kernel(x, W_m, W_t, b_t):
    t = x[:, 0]
    idx = x[:, 1].astype(jnp.int32)
    table = jnp.pad(W_m.T, ((0, _M_PAD - M_VOCAB), (0, 0)))
    return _sc_embed(t, idx, table, W_t, b_t)
